# baseline (speedup 1.0000x reference)
import functools

import jax
import jax.numpy as jnp
from jax import lax
from jax.experimental import pallas as pl
from jax.experimental.pallas import tpu as pltpu
from jax.experimental.pallas import tpu_sc as plsc

_HASH_MOD = 1000001
_LANES = 16
_PHASES = 16
_PAD = 32


def _pad_body(x_ref, o_ref):
    blk = x_ref.shape
    o_ref[...] = jnp.concatenate(
        [x_ref[...], jnp.zeros((blk[0], _PAD - blk[1]), jnp.int32)], axis=1
    )


def _unpad_body(x_ref, o_ref):
    o_ref[...] = x_ref[:, : o_ref.shape[1], :]


@functools.cache
def _build_pad(n_rows: int, n_cols: int):
    grid = 16
    blk = n_rows // grid
    return pl.pallas_call(
        _pad_body,
        grid=(grid,),
        in_specs=[pl.BlockSpec((blk, n_cols), lambda i: (i, 0))],
        out_specs=pl.BlockSpec((blk, _PAD), lambda i: (i, 0)),
        out_shape=jax.ShapeDtypeStruct((n_rows, _PAD), jnp.int32),
    )


@functools.cache
def _build_unpad(n_rows: int, n_cols: int, dim: int):
    grid = 64
    blk = n_rows // grid
    return pl.pallas_call(
        _unpad_body,
        grid=(grid,),
        in_specs=[pl.BlockSpec((blk, _PAD, dim), lambda i: (i, 0, 0))],
        out_specs=pl.BlockSpec((blk, n_cols, dim), lambda i: (i, 0, 0)),
        out_shape=jax.ShapeDtypeStruct((n_rows, n_cols, dim), jnp.float32),
    )


@functools.cache
def _build_gather(n_rows: int, dim: int, n_cols: int):
    assert _LANES <= n_cols <= _PAD
    info = plsc.get_sparse_core_info()
    nc, ns = info.num_cores, info.num_subcores
    nw = nc * ns
    assert n_rows % (nw * _PHASES) == 0
    rows_w = n_rows // nw
    rows_p = rows_w // _PHASES
    mesh = plsc.VectorSubcoreMesh(core_axis_name="c", subcore_axis_name="s")

    @functools.partial(
        pl.kernel,
        out_type=jax.ShapeDtypeStruct((n_rows, _PAD, dim), jnp.float32),
        mesh=mesh,
        compiler_params=pltpu.CompilerParams(use_tc_tiling_on_sc=False),
        scratch_types=[
            pltpu.VMEM((rows_w, _PAD), jnp.int32),
            pltpu.VMEM((2, rows_p, _PAD, dim), jnp.float32),
            pltpu.SemaphoreType.DMA,
            pltpu.SemaphoreType.DMA,
            pltpu.SemaphoreType.DMA,
        ],
    )
    def k(x_hbm, table_hbm, out_hbm, idx_v, rows_v, gsem, osem0, osem1):
        osem = (osem0, osem1)
        wid = lax.axis_index("s") * nc + lax.axis_index("c")
        r0 = wid * rows_w
        pltpu.sync_copy(x_hbm.at[pl.ds(r0, rows_w)], idx_v)

        def mod_phase(p):
            def body(i, carry):
                r = p * rows_p + i
                va = idx_v[r, pl.ds(0, _LANES)]
                idx_v[r, pl.ds(0, _LANES)] = lax.rem(
                    va, lax.full_like(va, _HASH_MOD)
                )
                vb = idx_v[r, pl.ds(n_cols - _LANES, _LANES)]
                idx_v[r, pl.ds(n_cols - _LANES, _LANES)] = lax.rem(
                    vb, lax.full_like(vb, _HASH_MOD)
                )
                return carry

            lax.fori_loop(0, rows_p, body, 0)

        def row_gather(p, b, i):
            return pltpu.make_async_copy(
                table_hbm.at[idx_v.at[p * rows_p + i].at[pl.ds(0, n_cols)]],
                rows_v.at[b, i, pl.ds(0, n_cols)],
                gsem,
            )

        def gather_start(p, b):
            lax.fori_loop(
                0, rows_p, lambda i, c: (row_gather(p, b, i).start(), c)[1], 0
            )

        def gather_wait(p, b):
            lax.fori_loop(
                0, rows_p, lambda i, c: (row_gather(p, b, i).wait(), c)[1], 0
            )

        def write_copy(p, b):
            return pltpu.make_async_copy(
                rows_v.at[b],
                out_hbm.at[pl.ds(r0 + p * rows_p, rows_p)],
                osem[b],
            )

        mod_phase(0)
        gather_start(0, 0)
        for p in range(_PHASES):
            b = p % 2
            if p + 1 < _PHASES:
                mod_phase(p + 1)
                gather_wait(p, b)
                if p >= 1:
                    write_copy(p - 1, 1 - b).wait()
                gather_start(p + 1, 1 - b)
            else:
                gather_wait(p, b)
            write_copy(p, b).start()
        write_copy(_PHASES - 2, _PHASES % 2).wait()
        write_copy(_PHASES - 1, (_PHASES - 1) % 2).wait()

    return k


def kernel(x, table):
    n_rows, n_cols = x.shape
    dim = table.shape[1]
    xp = _build_pad(n_rows, n_cols)(x)
    out = _build_gather(n_rows, dim, n_cols)(xp, table)
    return _build_unpad(n_rows, n_cols, dim)(out)


# speedup vs baseline: 1.1118x; 1.1118x over previous
import functools

import jax
import jax.numpy as jnp
from jax import lax
from jax.experimental import pallas as pl
from jax.experimental.pallas import tpu as pltpu
from jax.experimental.pallas import tpu_sc as plsc

_HASH_MOD = 1000001
_LANES = 16
_PHASES = 16
_PAD = 32


def _pad_body(x_ref, o_ref):
    blk = x_ref.shape
    o_ref[...] = jnp.concatenate(
        [x_ref[...], jnp.zeros((blk[0], _PAD - blk[1]), jnp.int32)], axis=1
    )


def _unpad_body(x_ref, o_ref):
    o_ref[...] = x_ref[:, : o_ref.shape[1], :]


@functools.cache
def _build_pad(n_rows: int, n_cols: int):
    grid = 16
    blk = n_rows // grid
    return pl.pallas_call(
        _pad_body,
        grid=(grid,),
        in_specs=[pl.BlockSpec((blk, n_cols), lambda i: (i, 0))],
        out_specs=pl.BlockSpec((blk, _PAD), lambda i: (i, 0)),
        out_shape=jax.ShapeDtypeStruct((n_rows, _PAD), jnp.int32),
    )


@functools.cache
def _build_unpad(n_rows: int, n_cols: int, dim: int):
    grid = 64
    blk = n_rows // grid
    return pl.pallas_call(
        _unpad_body,
        grid=(grid,),
        in_specs=[pl.BlockSpec((blk, _PAD, dim), lambda i: (i, 0, 0))],
        out_specs=pl.BlockSpec((blk, n_cols, dim), lambda i: (i, 0, 0)),
        out_shape=jax.ShapeDtypeStruct((n_rows, n_cols, dim), jnp.float32),
    )


@functools.cache
def _build_gather(n_rows: int, dim: int, n_cols: int):
    assert _LANES <= n_cols <= _PAD
    info = plsc.get_sparse_core_info()
    nc, ns = info.num_cores, info.num_subcores
    nw = nc * ns
    assert n_rows % (nw * _PHASES) == 0
    rows_w = n_rows // nw
    rows_p = rows_w // _PHASES
    mesh = plsc.VectorSubcoreMesh(core_axis_name="c", subcore_axis_name="s")

    @functools.partial(
        pl.kernel,
        out_type=jax.ShapeDtypeStruct((n_rows, _PAD, dim), jnp.float32),
        mesh=mesh,
        compiler_params=pltpu.CompilerParams(use_tc_tiling_on_sc=False),
        scratch_types=[
            pltpu.VMEM((rows_w, _PAD), jnp.int32),
            pltpu.VMEM((2, rows_p, _PAD, dim), jnp.float32),
            pltpu.SemaphoreType.DMA,
            pltpu.SemaphoreType.DMA,
            pltpu.SemaphoreType.DMA,
        ],
    )
    def k(x_hbm, table_hbm, out_hbm, idx_v, rows_v, gsem, osem0, osem1):
        osem = (osem0, osem1)
        wid = lax.axis_index("s") * nc + lax.axis_index("c")
        r0 = wid * rows_w
        pltpu.sync_copy(x_hbm.at[pl.ds(r0, rows_w)], idx_v)

        def mod_phase(p):
            def body(i, carry):
                r = p * rows_p + i
                va = idx_v[r, pl.ds(0, _LANES)]
                idx_v[r, pl.ds(0, _LANES)] = lax.rem(
                    va, lax.full_like(va, _HASH_MOD)
                )
                vb = idx_v[r, pl.ds(n_cols - _LANES, _LANES)]
                idx_v[r, pl.ds(n_cols - _LANES, _LANES)] = lax.rem(
                    vb, lax.full_like(vb, _HASH_MOD)
                )
                return carry

            lax.fori_loop(0, rows_p, body, 0)

        def row_gather(p, b, i):
            return pltpu.make_async_copy(
                table_hbm.at[idx_v.at[p * rows_p + i].at[pl.ds(0, n_cols)]],
                rows_v.at[b, i, pl.ds(0, n_cols)],
                gsem,
            )

        def gather_start(p, b):
            lax.fori_loop(
                0, rows_p, lambda i, c: (row_gather(p, b, i).start(), c)[1], 0
            )

        def gather_wait(p, b):
            lax.fori_loop(
                0, rows_p, lambda i, c: (row_gather(p, b, i).wait(), c)[1], 0
            )

        def write_copy(p, b):
            return pltpu.make_async_copy(
                rows_v.at[b],
                out_hbm.at[pl.ds(r0 + p * rows_p, rows_p)],
                osem[b],
            )

        mod_phase(0)
        gather_start(0, 0)
        for p in range(_PHASES):
            b = p % 2
            if p + 1 < _PHASES:
                mod_phase(p + 1)
                gather_wait(p, b)
                if p >= 1:
                    write_copy(p - 1, 1 - b).wait()
                gather_start(p + 1, 1 - b)
            else:
                gather_wait(p, b)
            write_copy(p, b).start()
        write_copy(_PHASES - 2, _PHASES % 2).wait()
        write_copy(_PHASES - 1, (_PHASES - 1) % 2).wait()

    return k


def kernel(x, table):
    n_rows, n_cols = x.shape
    dim = table.shape[1]
    zero_f = table[0, 0] * 0.0
    zero_i = zero_f.astype(jnp.int32)
    xp = jnp.pad(x, ((0, 0), (0, _PAD - n_cols))) + zero_i
    out = _build_gather(n_rows, dim, n_cols)(xp, table)
    return out[:, :n_cols, :] + zero_f


# speedup vs baseline: 1.3411x; 1.2063x over previous
import functools

import jax
import jax.numpy as jnp
from jax import lax
from jax.experimental import pallas as pl
from jax.experimental.pallas import tpu as pltpu
from jax.experimental.pallas import tpu_sc as plsc

_HASH_MOD = 1000001
_LANES = 16
_PHASES = 8
_PAD = 32


def _pad_body(x_ref, o_ref):
    blk = x_ref.shape
    o_ref[...] = jnp.concatenate(
        [x_ref[...], jnp.zeros((blk[0], _PAD - blk[1]), jnp.int32)], axis=1
    )


@functools.cache
def _build_pad(n_rows: int, n_cols: int):
    grid = 16
    blk = n_rows // grid
    return pl.pallas_call(
        _pad_body,
        grid=(grid,),
        in_specs=[pl.BlockSpec((blk, n_cols), lambda i: (i, 0))],
        out_specs=pl.BlockSpec((blk, _PAD), lambda i: (i, 0)),
        out_shape=jax.ShapeDtypeStruct((n_rows, _PAD), jnp.int32),
    )


@functools.cache
def _build_gather(n_rows: int, dim: int, n_cols: int):
    assert _LANES <= n_cols <= _PAD
    info = plsc.get_sparse_core_info()
    nc, ns = info.num_cores, info.num_subcores
    nw = nc * ns
    assert n_rows % (nw * _PHASES) == 0
    rows_w = n_rows // nw
    rows_p = rows_w // _PHASES
    mesh = plsc.VectorSubcoreMesh(core_axis_name="c", subcore_axis_name="s")

    @functools.partial(
        pl.kernel,
        out_type=jax.ShapeDtypeStruct((n_rows, n_cols, dim), jnp.float32),
        mesh=mesh,
        compiler_params=pltpu.CompilerParams(use_tc_tiling_on_sc=False),
        scratch_types=[
            pltpu.VMEM((rows_w, _PAD), jnp.int32),
            pltpu.VMEM((2, rows_p, n_cols, dim), jnp.float32),
            pltpu.SemaphoreType.DMA,
            pltpu.SemaphoreType.DMA,
            pltpu.SemaphoreType.DMA,
        ],
    )
    def k(x_hbm, table_hbm, out_hbm, idx_v, rows_v, gsem, osem0, osem1):
        osem = (osem0, osem1)
        wid = lax.axis_index("s") * nc + lax.axis_index("c")
        r0 = wid * rows_w
        pltpu.sync_copy(x_hbm.at[pl.ds(r0, rows_w)], idx_v)

        def mod_phase(p):
            def body(i, carry):
                r = p * rows_p + i
                va = idx_v[r, pl.ds(0, _LANES)]
                idx_v[r, pl.ds(0, _LANES)] = lax.rem(
                    va, lax.full_like(va, _HASH_MOD)
                )
                vb = idx_v[r, pl.ds(n_cols - _LANES, _LANES)]
                idx_v[r, pl.ds(n_cols - _LANES, _LANES)] = lax.rem(
                    vb, lax.full_like(vb, _HASH_MOD)
                )
                return carry

            lax.fori_loop(0, rows_p, body, 0)

        def row_gather(p, b, i):
            return pltpu.make_async_copy(
                table_hbm.at[idx_v.at[p * rows_p + i].at[pl.ds(0, n_cols)]],
                rows_v.at[b, i],
                gsem,
            )

        def gather_start(p, b):
            lax.fori_loop(
                0, rows_p, lambda i, c: (row_gather(p, b, i).start(), c)[1], 0
            )

        def gather_wait(p, b):
            lax.fori_loop(
                0, rows_p, lambda i, c: (row_gather(p, b, i).wait(), c)[1], 0
            )

        def write_copy(p, b):
            return pltpu.make_async_copy(
                rows_v.at[b],
                out_hbm.at[pl.ds(r0 + p * rows_p, rows_p)],
                osem[b],
            )

        mod_phase(0)
        gather_start(0, 0)
        for p in range(_PHASES):
            b = p % 2
            if p + 1 < _PHASES:
                mod_phase(p + 1)
                gather_wait(p, b)
                if p >= 1:
                    write_copy(p - 1, 1 - b).wait()
                gather_start(p + 1, 1 - b)
            else:
                gather_wait(p, b)
            write_copy(p, b).start()
        write_copy(_PHASES - 2, _PHASES % 2).wait()
        write_copy(_PHASES - 1, (_PHASES - 1) % 2).wait()

    return k


def kernel(x, table):
    n_rows, n_cols = x.shape
    dim = table.shape[1]
    xp = _build_pad(n_rows, n_cols)(x)
    return _build_gather(n_rows, dim, n_cols)(xp, table)


# speedup vs baseline: 1.3651x; 1.0179x over previous
import functools

import jax
import jax.numpy as jnp
from jax import lax
from jax.experimental import pallas as pl
from jax.experimental.pallas import tpu as pltpu
from jax.experimental.pallas import tpu_sc as plsc

_HASH_MOD = 1000001
_LANES = 16
_CHUNK = 1664
_PHASES = 8


@functools.cache
def _build(n_total: int, dim: int):
    info = plsc.get_sparse_core_info()
    nc, ns = info.num_cores, info.num_subcores
    nw = nc * ns
    assert n_total % nw == 0
    per_w = n_total // nw
    assert per_w == _CHUNK * _PHASES
    mesh = plsc.VectorSubcoreMesh(core_axis_name="c", subcore_axis_name="s")

    @functools.partial(
        pl.kernel,
        out_type=jax.ShapeDtypeStruct((n_total, dim), jnp.float32),
        mesh=mesh,
        compiler_params=pltpu.CompilerParams(use_tc_tiling_on_sc=False),
        scratch_types=[
            pltpu.VMEM((per_w,), jnp.int32),
            pltpu.VMEM((2, _CHUNK, dim), jnp.float32),
            pltpu.SemaphoreType.DMA,
            pltpu.SemaphoreType.DMA,
            pltpu.SemaphoreType.DMA,
        ],
    )
    def k(x_hbm, table_hbm, out_hbm, idx_v, rows_v, gsem, osem0, osem1):
        osem = (osem0, osem1)
        wid = lax.axis_index("s") * nc + lax.axis_index("c")
        base = wid * per_w
        pltpu.sync_copy(x_hbm.at[pl.ds(base, per_w)], idx_v)

        def mod_phase(p):
            def body(j, carry):
                sl = pl.ds(p * _CHUNK + j * _LANES, _LANES)
                v = idx_v[sl]
                idx_v[sl] = lax.rem(v, lax.full_like(v, _HASH_MOD))
                return carry

            lax.fori_loop(0, _CHUNK // _LANES, body, 0)

        def gather_copy(p, b):
            return pltpu.make_async_copy(
                table_hbm.at[idx_v.at[pl.ds(p * _CHUNK, _CHUNK)]],
                rows_v.at[b],
                gsem,
            )

        def write_copy(p, b):
            return pltpu.make_async_copy(
                rows_v.at[b],
                out_hbm.at[pl.ds(base + p * _CHUNK, _CHUNK)],
                osem[b],
            )

        mod_phase(0)
        gather_copy(0, 0).start()
        for p in range(_PHASES):
            b = p % 2
            if p + 1 < _PHASES:
                mod_phase(p + 1)
                gather_copy(p, b).wait()
                if p >= 1:
                    write_copy(p - 1, 1 - b).wait()
                gather_copy(p + 1, 1 - b).start()
            else:
                gather_copy(p, b).wait()
            write_copy(p, b).start()
        write_copy(_PHASES - 2, _PHASES % 2).wait()
        write_copy(_PHASES - 1, (_PHASES - 1) % 2).wait()

    return k


def kernel(x, table):
    n_total = x.size
    out = _build(n_total, table.shape[1])(x.reshape(n_total), table)
    return out.reshape(*x.shape, table.shape[1])
